# Initial kernel scaffold; baseline (speedup 1.0000x reference)
import jax
import jax.numpy as jnp
from jax.experimental import pallas as pl
from jax.experimental.pallas import tpu as pltpu
from jax.experimental.pallas import tpu_sc as plsc

_B = 16384 * 26
_D = 32
_W = 128


def kernel(x, weight):
    idx = x.reshape(1, _B)
    mesh = plsc.VectorSubcoreMesh(core_axis_name="core", subcore_axis_name="subcore")

    @pl.kernel(out_type=jax.ShapeDtypeStruct((_B, _D), weight.dtype), mesh=mesh)
    def gather_kernel(w_hbm, i_hbm, o_hbm):
        def body(i_vmem, o_vmem):
            pltpu.sync_copy(w_hbm.at[i_vmem.at[0]], o_vmem)

        pltpu.emit_pipeline(
            body,
            grid=(_B // _W,),
            in_specs=[pl.BlockSpec((1, _W), index_map=lambda i: (0, i))],
            out_specs=[pl.BlockSpec((_W, _D), index_map=lambda i: (i, 0))],
            core_axis_name=("core", "subcore"),
            dimension_semantics=(pltpu.PARALLEL,),
        )(i_hbm, o_hbm)

    out = gather_kernel(weight, idx)
    return out.reshape(x.shape[0], x.shape[1], _D)


# speedup vs baseline: 1.5254x; 1.5254x over previous
import functools

import jax
import jax.numpy as jnp
from jax import lax
from jax.experimental import pallas as pl
from jax.experimental.pallas import tpu as pltpu
from jax.experimental.pallas import tpu_sc as plsc

_B = 16384 * 26
_D = 32
_C = 128
_NC, _NS = 2, 16
_NW = _NC * _NS
_PER_W = _B // _NW
_NCHUNK = _PER_W // _C


def kernel(x, weight):
    idx = x.reshape(_NW, _NCHUNK, _C)
    mesh = plsc.VectorSubcoreMesh(core_axis_name="c", subcore_axis_name="s")

    @functools.partial(
        pl.kernel,
        mesh=mesh,
        out_type=jax.ShapeDtypeStruct((_B, _D), jnp.float32),
        compiler_params=pltpu.CompilerParams(use_tc_tiling_on_sc=False),
        scratch_types=[
            pltpu.VMEM((_NCHUNK, _C), jnp.int32),
            pltpu.VMEM((_C, _D), jnp.float32),
            pltpu.VMEM((_C, _D), jnp.float32),
            pltpu.SemaphoreType.DMA,
            pltpu.SemaphoreType.DMA,
        ],
    )
    def gather_kernel(w_hbm, i_hbm, o_hbm, idx_v, rows0, rows1, sem0, sem1):
        wid = lax.axis_index("s") * _NC + lax.axis_index("c")
        base = wid * _PER_W

        pltpu.sync_copy(i_hbm.at[wid], idx_v)

        def start(j, rows, sem):
            pltpu.async_copy(w_hbm.at[idx_v.at[j]], rows, sem)

        def wait(rows, sem):
            pltpu.make_async_copy(w_hbm.at[pl.ds(0, _C)], rows, sem).wait()

        def store(j, rows):
            pltpu.sync_copy(rows, o_hbm.at[pl.ds(base + j * _C, _C)])

        start(0, rows0, sem0)
        start(1, rows1, sem1)

        @pl.loop(0, _NCHUNK - 2, step=2)
        def _(j):
            wait(rows0, sem0)
            store(j, rows0)
            start(j + 2, rows0, sem0)
            wait(rows1, sem1)
            store(j + 1, rows1)
            start(j + 3, rows1, sem1)

        wait(rows0, sem0)
        store(_NCHUNK - 2, rows0)
        wait(rows1, sem1)
        store(_NCHUNK - 1, rows1)

    out = gather_kernel(weight, idx)
    return out.reshape(x.shape[0], x.shape[1], _D)
